# baseline (speedup 1.0000x reference)
import functools

import jax
import jax.numpy as jnp
from jax.experimental import pallas as pl
from jax.experimental.pallas import tpu as pltpu


def _pos_kernel(col_ref, row_ref, out_hbm, scratch, sems, *, h, w, B):
    _, d = col_ref.shape

    col_t = col_ref[0:w, :].T
    row_t = row_ref[0:h, :].T
    scratch[0:d] = jnp.broadcast_to(col_t[:, None, :], (d, h, w))
    scratch[d : 2 * d] = jnp.broadcast_to(row_t[:, :, None], (d, h, w))

    for b in range(B):
        pltpu.make_async_copy(scratch, out_hbm.at[b], sems.at[b]).start()
    for b in range(B):
        pltpu.make_async_copy(scratch, out_hbm.at[b], sems.at[b]).wait()


def kernel(x, mask, row_embed, col_embed):
    B = x.shape[0]
    h, w = x.shape[-2], x.shape[-1]
    n, d = col_embed.shape

    return pl.pallas_call(
        functools.partial(_pos_kernel, h=h, w=w, B=B),
        in_specs=[
            pl.BlockSpec(memory_space=pltpu.MemorySpace.VMEM),
            pl.BlockSpec(memory_space=pltpu.MemorySpace.VMEM),
        ],
        out_specs=pl.BlockSpec(memory_space=pl.ANY),
        out_shape=jax.ShapeDtypeStruct((B, 2 * d, h, w), jnp.float32),
        scratch_shapes=[
            pltpu.VMEM((2 * d, h, w), jnp.float32),
            pltpu.SemaphoreType.DMA((B,)),
        ],
    )(col_embed, row_embed)
